# baseline (speedup 1.0000x reference)
import functools

import jax
import jax.numpy as jnp
from jax import lax
from jax.experimental import pallas as pl
from jax.experimental.pallas import tpu as pltpu
from jax.experimental.pallas import tpu_sc as plsc

N = 10000
D = 128
H = 128
NP = 10240
TRASH = 10000
NC = 2
NS = 16
L = 16
NW = NC * NS
CH = 128
CPT = 80
EP = NW * CPT * CH
NCH = EP // CH
BLK = 1024
TG = NP // BLK
NPS = NP // NS

_mesh = plsc.VectorSubcoreMesh(
    core_axis_name="c", subcore_axis_name="s", num_cores=NC, num_subcores=NS
)


@functools.partial(
    pl.kernel,
    out_type=jax.ShapeDtypeStruct((NC, NP), jnp.float32),
    mesh=_mesh,
    scratch_types=[
        pltpu.VMEM((CPT, CH), jnp.int32),
        pltpu.VMEM((CH,), jnp.float32),
        pltpu.VMEM_SHARED((NP,), jnp.float32),
    ],
)
def _deg_kernel(dst2d, zeros_np, deg_out, didx, ones, deg_sp):
    cid = lax.axis_index("c")
    sid = lax.axis_index("s")
    wid = cid * NS + sid
    for j in range(CH // L):
        ones[pl.ds(j * L, L)] = jnp.ones((L,), jnp.float32)
    pltpu.sync_copy(dst2d.at[pl.ds(wid * CPT, CPT)], didx)

    @pl.when(sid == 0)
    def _():
        pltpu.sync_copy(zeros_np, deg_sp)

    plsc.subcore_barrier()

    def body(k, carry):
        pltpu.sync_copy(ones, deg_sp.at[didx.at[k]], add=True)
        return carry

    lax.fori_loop(0, CPT, body, 0)
    plsc.subcore_barrier()
    pltpu.sync_copy(
        deg_sp.at[pl.ds(sid * NPS, NPS)],
        deg_out.at[cid, pl.ds(sid * NPS, NPS)],
    )


GRP = 16
NGA = NCH // NS // GRP


@functools.partial(
    pl.kernel,
    out_type=jax.ShapeDtypeStruct((NP, H), jnp.float32),
    mesh=_mesh,
    scratch_types=[
        pltpu.VMEM((GRP, CH), jnp.int32),
        pltpu.VMEM((GRP, CH), jnp.int32),
        pltpu.VMEM((2, CH, H), jnp.float32),
        pltpu.VMEM_SHARED((NP, H), jnp.float32),
        pltpu.SemaphoreType.DMA,
        pltpu.SemaphoreType.DMA,
    ],
)
def _agg_kernel(src2d, dst2d, u_hbm, zeros_nph, acc_out,
                sidx, didx, rows, acc_sp, sem0, sem1):
    cid = lax.axis_index("c")
    sid = lax.axis_index("s")
    ng = jnp.where(cid == 0, NGA, 0)
    start = sid * (NGA * GRP)

    @pl.when(cid == 0)
    def _():
        pltpu.sync_copy(
            zeros_nph.at[pl.ds(sid * NPS, NPS)],
            acc_sp.at[pl.ds(sid * NPS, NPS)],
        )

    plsc.subcore_barrier()

    def gbody(g, carry):
        base = start + g * GRP
        pltpu.sync_copy(src2d.at[pl.ds(base, GRP)], sidx)
        pltpu.sync_copy(dst2d.at[pl.ds(base, GRP)], didx)
        pltpu.async_copy(u_hbm.at[sidx.at[0]], rows.at[0], sem0)

        def body(i, c2):
            k0 = i * 2
            pltpu.async_copy(u_hbm.at[sidx.at[k0 + 1]], rows.at[1], sem1)
            pltpu.make_async_copy(
                u_hbm.at[sidx.at[k0]], rows.at[0], sem0).wait()
            pltpu.sync_copy(rows.at[0], acc_sp.at[didx.at[k0]], add=True)

            @pl.when(k0 + 2 < GRP)
            def _():
                pltpu.async_copy(u_hbm.at[sidx.at[k0 + 2]], rows.at[0], sem0)

            pltpu.make_async_copy(
                u_hbm.at[sidx.at[k0 + 1]], rows.at[1], sem1).wait()
            pltpu.sync_copy(rows.at[1], acc_sp.at[didx.at[k0 + 1]], add=True)
            return c2

        lax.fori_loop(0, GRP // 2, body, 0)
        return carry

    lax.fori_loop(0, ng, gbody, 0)
    plsc.subcore_barrier()

    @pl.when(cid == 0)
    def _():
        pltpu.sync_copy(
            acc_sp.at[pl.ds(sid * NPS, NPS)],
            acc_out.at[pl.ds(sid * NPS, NPS)],
        )


@functools.partial(
    pl.kernel,
    out_type=jax.ShapeDtypeStruct((NC, NP), jnp.float32),
    mesh=_mesh,
    scratch_types=[
        pltpu.VMEM((CPT, CH), jnp.int32),
        pltpu.VMEM((CPT, CH), jnp.int32),
        pltpu.VMEM((2, CH), jnp.float32),
        pltpu.VMEM_SHARED((NP,), jnp.float32),
        pltpu.SemaphoreType.DMA,
        pltpu.SemaphoreType.DMA,
    ],
)
def _agg2_kernel(src2d, dst2d, v_hbm, zeros_np, agg_out,
                 sidx, didx, vals, agg_sp, sem0, sem1):
    cid = lax.axis_index("c")
    sid = lax.axis_index("s")
    wid = cid * NS + sid
    pltpu.sync_copy(src2d.at[pl.ds(wid * CPT, CPT)], sidx)
    pltpu.sync_copy(dst2d.at[pl.ds(wid * CPT, CPT)], didx)

    @pl.when(sid == 0)
    def _():
        pltpu.sync_copy(zeros_np, agg_sp)

    plsc.subcore_barrier()
    pltpu.async_copy(v_hbm.at[sidx.at[0]], vals.at[0], sem0)

    def body(i, carry):
        k0 = i * 2
        pltpu.async_copy(v_hbm.at[sidx.at[k0 + 1]], vals.at[1], sem1)
        pltpu.make_async_copy(
            v_hbm.at[sidx.at[k0]], vals.at[0], sem0).wait()
        pltpu.sync_copy(vals.at[0], agg_sp.at[didx.at[k0]], add=True)

        @pl.when(k0 + 2 < CPT)
        def _():
            pltpu.async_copy(v_hbm.at[sidx.at[k0 + 2]], vals.at[0], sem0)

        pltpu.make_async_copy(
            v_hbm.at[sidx.at[k0 + 1]], vals.at[1], sem1).wait()
        pltpu.sync_copy(vals.at[1], agg_sp.at[didx.at[k0 + 1]], add=True)
        return carry

    lax.fori_loop(0, CPT // 2, body, 0)
    plsc.subcore_barrier()
    pltpu.sync_copy(
        agg_sp.at[pl.ds(sid * NPS, NPS)],
        agg_out.at[cid, pl.ds(sid * NPS, NPS)],
    )


def _mm1_body(x_ref, w_ref, d0_ref, d1_ref, u_ref, dinv_ref):
    h = jnp.dot(x_ref[...], w_ref[...], preferred_element_type=jnp.float32)
    deg = d0_ref[...] + d1_ref[...] + 1.0
    dinv = lax.rsqrt(deg)
    u_ref[...] = h * dinv
    dinv_ref[...] = dinv


_mm1 = pl.pallas_call(
    _mm1_body,
    grid=(TG,),
    in_specs=[
        pl.BlockSpec((BLK, D), lambda i: (i, 0)),
        pl.BlockSpec((D, H), lambda i: (0, 0)),
        pl.BlockSpec((BLK, 1), lambda i: (i, 0)),
        pl.BlockSpec((BLK, 1), lambda i: (i, 0)),
    ],
    out_specs=[
        pl.BlockSpec((BLK, H), lambda i: (i, 0)),
        pl.BlockSpec((BLK, 1), lambda i: (i, 0)),
    ],
    out_shape=[
        jax.ShapeDtypeStruct((NP, H), jnp.float32),
        jax.ShapeDtypeStruct((NP, 1), jnp.float32),
    ],
)


def _mm2_body(a0_ref, u_ref, dinv_ref, w2_ref, b1_ref, v_ref):
    dinv = dinv_ref[...]
    out1 = dinv * (a0_ref[...] + u_ref[...]) + b1_ref[...]
    a = jnp.maximum(out1, 0.0)
    z = jnp.dot(a, w2_ref[...], preferred_element_type=jnp.float32)
    v_ref[...] = z * dinv


_mm2 = pl.pallas_call(
    _mm2_body,
    grid=(TG,),
    in_specs=[
        pl.BlockSpec((BLK, H), lambda i: (i, 0)),
        pl.BlockSpec((BLK, H), lambda i: (i, 0)),
        pl.BlockSpec((BLK, 1), lambda i: (i, 0)),
        pl.BlockSpec((H, 1), lambda i: (0, 0)),
        pl.BlockSpec((1, H), lambda i: (0, 0)),
    ],
    out_specs=pl.BlockSpec((BLK, 1), lambda i: (i, 0)),
    out_shape=jax.ShapeDtypeStruct((NP, 1), jnp.float32),
)


def _fin_body(a0_ref, a1_ref, v_ref, dinv_ref, b2_ref, o_ref):
    o_ref[...] = (
        dinv_ref[...] * (a0_ref[...] + a1_ref[...] + v_ref[...]) + b2_ref[...]
    )


_fin = pl.pallas_call(
    _fin_body,
    grid=(TG,),
    in_specs=[
        pl.BlockSpec((BLK, 1), lambda i: (i, 0)),
        pl.BlockSpec((BLK, 1), lambda i: (i, 0)),
        pl.BlockSpec((BLK, 1), lambda i: (i, 0)),
        pl.BlockSpec((BLK, 1), lambda i: (i, 0)),
        pl.BlockSpec((1, 1), lambda i: (0, 0)),
    ],
    out_specs=pl.BlockSpec((BLK, 1), lambda i: (i, 0)),
    out_shape=jax.ShapeDtypeStruct((NP, 1), jnp.float32),
)


def kernel(x, edge_index, W1, b1, W2, b2):
    e = edge_index.shape[1]
    src = edge_index[0].astype(jnp.int32)
    dst = edge_index[1].astype(jnp.int32)
    src_p = jnp.concatenate([src, jnp.zeros((EP - e,), jnp.int32)])
    pad_dst = TRASH + jax.lax.rem(
        jnp.arange(EP - e, dtype=jnp.int32), jnp.int32(NP - N)
    )
    dst_p = jnp.concatenate([dst, pad_dst])
    src2d = src_p.reshape(NCH, CH)
    dst2d = dst_p.reshape(NCH, CH)
    zeros_np = jnp.zeros((NP,), jnp.float32)
    zeros_nph = jnp.zeros((NP, H), jnp.float32)
    x_p = jnp.concatenate(
        [x.astype(jnp.float32), jnp.zeros((NP - N, D), jnp.float32)]
    )

    degp = _deg_kernel(dst2d, zeros_np)
    deg0 = degp[0].reshape(NP, 1)
    deg1 = degp[1].reshape(NP, 1)
    u, dinv = _mm1(x_p, W1, deg0, deg1)
    acc = _agg_kernel(src2d, dst2d, u, zeros_nph)
    v = _mm2(acc, u, dinv, W2, b1.reshape(1, H))
    aggp = _agg2_kernel(src2d, dst2d, v.reshape(NP), zeros_np)
    out = _fin(
        aggp[0].reshape(NP, 1), aggp[1].reshape(NP, 1), v, dinv,
        b2.reshape(1, 1),
    )
    return out[:N]


# speedup vs baseline: 1.0064x; 1.0064x over previous
import functools

import jax
import jax.numpy as jnp
from jax import lax
from jax.experimental import pallas as pl
from jax.experimental.pallas import tpu as pltpu
from jax.experimental.pallas import tpu_sc as plsc

N = 10000
D = 128
H = 128
NP = 10240
TRASH = 10000
NC = 2
NS = 16
L = 16
NW = NC * NS
CH = 128
CPT = 80
EP = NW * CPT * CH
NCH = EP // CH
BLK = 1024
TG = NP // BLK
NPS = NP // NS

_mesh = plsc.VectorSubcoreMesh(
    core_axis_name="c", subcore_axis_name="s", num_cores=NC, num_subcores=NS
)


@functools.partial(
    pl.kernel,
    out_type=jax.ShapeDtypeStruct((NC, NP), jnp.float32),
    mesh=_mesh,
    scratch_types=[
        pltpu.VMEM((CPT, CH), jnp.int32),
        pltpu.VMEM((CH,), jnp.float32),
        pltpu.VMEM_SHARED((NP,), jnp.float32),
    ],
)
def _deg_kernel(dst2d, zeros_np, deg_out, didx, ones, deg_sp):
    cid = lax.axis_index("c")
    sid = lax.axis_index("s")
    wid = cid * NS + sid
    for j in range(CH // L):
        ones[pl.ds(j * L, L)] = jnp.ones((L,), jnp.float32)
    pltpu.sync_copy(dst2d.at[pl.ds(wid * CPT, CPT)], didx)

    @pl.when(sid == 0)
    def _():
        pltpu.sync_copy(zeros_np, deg_sp)

    plsc.subcore_barrier()

    def body(k, carry):
        pltpu.sync_copy(ones, deg_sp.at[didx.at[k]], add=True)
        return carry

    lax.fori_loop(0, CPT, body, 0)
    plsc.subcore_barrier()
    pltpu.sync_copy(
        deg_sp.at[pl.ds(sid * NPS, NPS)],
        deg_out.at[cid, pl.ds(sid * NPS, NPS)],
    )


GRP = 16
NGA = NCH // NS // GRP


@functools.partial(
    pl.kernel,
    out_type=jax.ShapeDtypeStruct((NP, H), jnp.float32),
    mesh=_mesh,
    scratch_types=[
        pltpu.VMEM((GRP, CH), jnp.int32),
        pltpu.VMEM((GRP, CH), jnp.int32),
        pltpu.VMEM((2, CH, H), jnp.float32),
        pltpu.VMEM_SHARED((NP, H), jnp.float32),
        pltpu.SemaphoreType.DMA,
        pltpu.SemaphoreType.DMA,
    ],
)
def _agg_kernel(src2d, dst2d, u_hbm, acc_out,
                sidx, didx, rows, acc_sp, sem0, sem1):
    cid = lax.axis_index("c")
    sid = lax.axis_index("s")
    ng = jnp.where(cid == 0, NGA, 0)
    start = sid * (NGA * GRP)

    @pl.when((cid == 0) & (sid == 0))
    def _():
        pltpu.sync_copy(u_hbm, acc_sp)

    plsc.subcore_barrier()

    def gbody(g, carry):
        base = start + g * GRP
        pltpu.sync_copy(src2d.at[pl.ds(base, GRP)], sidx)
        pltpu.sync_copy(dst2d.at[pl.ds(base, GRP)], didx)
        pltpu.async_copy(u_hbm.at[sidx.at[0]], rows.at[0], sem0)

        def body(i, c2):
            k0 = i * 2
            pltpu.async_copy(u_hbm.at[sidx.at[k0 + 1]], rows.at[1], sem1)
            pltpu.make_async_copy(
                u_hbm.at[sidx.at[k0]], rows.at[0], sem0).wait()
            pltpu.sync_copy(rows.at[0], acc_sp.at[didx.at[k0]], add=True)

            @pl.when(k0 + 2 < GRP)
            def _():
                pltpu.async_copy(u_hbm.at[sidx.at[k0 + 2]], rows.at[0], sem0)

            pltpu.make_async_copy(
                u_hbm.at[sidx.at[k0 + 1]], rows.at[1], sem1).wait()
            pltpu.sync_copy(rows.at[1], acc_sp.at[didx.at[k0 + 1]], add=True)
            return c2

        lax.fori_loop(0, GRP // 2, body, 0)
        return carry

    lax.fori_loop(0, ng, gbody, 0)
    plsc.subcore_barrier()

    @pl.when(cid == 0)
    def _():
        pltpu.sync_copy(
            acc_sp.at[pl.ds(sid * NPS, NPS)],
            acc_out.at[pl.ds(sid * NPS, NPS)],
        )


@functools.partial(
    pl.kernel,
    out_type=jax.ShapeDtypeStruct((NC, NP), jnp.float32),
    mesh=_mesh,
    scratch_types=[
        pltpu.VMEM((CPT, CH), jnp.int32),
        pltpu.VMEM((CPT, CH), jnp.int32),
        pltpu.VMEM((2, CH), jnp.float32),
        pltpu.VMEM_SHARED((NP,), jnp.float32),
        pltpu.SemaphoreType.DMA,
        pltpu.SemaphoreType.DMA,
    ],
)
def _agg2_kernel(src2d, dst2d, v_hbm, zeros_np, agg_out,
                 sidx, didx, vals, agg_sp, sem0, sem1):
    cid = lax.axis_index("c")
    sid = lax.axis_index("s")
    wid = cid * NS + sid
    pltpu.sync_copy(src2d.at[pl.ds(wid * CPT, CPT)], sidx)
    pltpu.sync_copy(dst2d.at[pl.ds(wid * CPT, CPT)], didx)

    @pl.when(sid == 0)
    def _():
        pltpu.sync_copy(zeros_np, agg_sp)

    plsc.subcore_barrier()
    pltpu.async_copy(v_hbm.at[sidx.at[0]], vals.at[0], sem0)

    def body(i, carry):
        k0 = i * 2
        pltpu.async_copy(v_hbm.at[sidx.at[k0 + 1]], vals.at[1], sem1)
        pltpu.make_async_copy(
            v_hbm.at[sidx.at[k0]], vals.at[0], sem0).wait()
        pltpu.sync_copy(vals.at[0], agg_sp.at[didx.at[k0]], add=True)

        @pl.when(k0 + 2 < CPT)
        def _():
            pltpu.async_copy(v_hbm.at[sidx.at[k0 + 2]], vals.at[0], sem0)

        pltpu.make_async_copy(
            v_hbm.at[sidx.at[k0 + 1]], vals.at[1], sem1).wait()
        pltpu.sync_copy(vals.at[1], agg_sp.at[didx.at[k0 + 1]], add=True)
        return carry

    lax.fori_loop(0, CPT // 2, body, 0)
    plsc.subcore_barrier()
    pltpu.sync_copy(
        agg_sp.at[pl.ds(sid * NPS, NPS)],
        agg_out.at[cid, pl.ds(sid * NPS, NPS)],
    )


def _mm1_body(x_ref, w_ref, d0_ref, d1_ref, u_ref, dinv_ref):
    h = jnp.dot(x_ref[...], w_ref[...], preferred_element_type=jnp.float32)
    deg = d0_ref[...] + d1_ref[...] + 1.0
    dinv = lax.rsqrt(deg)
    u_ref[...] = h * dinv
    dinv_ref[...] = dinv


_mm1 = pl.pallas_call(
    _mm1_body,
    grid=(TG,),
    in_specs=[
        pl.BlockSpec((BLK, D), lambda i: (i, 0)),
        pl.BlockSpec((D, H), lambda i: (0, 0)),
        pl.BlockSpec((BLK, 1), lambda i: (i, 0)),
        pl.BlockSpec((BLK, 1), lambda i: (i, 0)),
    ],
    out_specs=[
        pl.BlockSpec((BLK, H), lambda i: (i, 0)),
        pl.BlockSpec((BLK, 1), lambda i: (i, 0)),
    ],
    out_shape=[
        jax.ShapeDtypeStruct((NP, H), jnp.float32),
        jax.ShapeDtypeStruct((NP, 1), jnp.float32),
    ],
)


def _mm2_body(a0_ref, dinv_ref, w2_ref, b1_ref, v_ref):
    dinv = dinv_ref[...]
    out1 = dinv * a0_ref[...] + b1_ref[...]
    a = jnp.maximum(out1, 0.0)
    z = jnp.dot(a, w2_ref[...], preferred_element_type=jnp.float32)
    v_ref[...] = z * dinv


_mm2 = pl.pallas_call(
    _mm2_body,
    grid=(TG,),
    in_specs=[
        pl.BlockSpec((BLK, H), lambda i: (i, 0)),
        pl.BlockSpec((BLK, 1), lambda i: (i, 0)),
        pl.BlockSpec((H, 1), lambda i: (0, 0)),
        pl.BlockSpec((1, H), lambda i: (0, 0)),
    ],
    out_specs=pl.BlockSpec((BLK, 1), lambda i: (i, 0)),
    out_shape=jax.ShapeDtypeStruct((NP, 1), jnp.float32),
)


def _fin_body(a0_ref, a1_ref, v_ref, dinv_ref, b2_ref, o_ref):
    o_ref[...] = (
        dinv_ref[...] * (a0_ref[...] + a1_ref[...] + v_ref[...]) + b2_ref[...]
    )


_fin = pl.pallas_call(
    _fin_body,
    grid=(TG,),
    in_specs=[
        pl.BlockSpec((BLK, 1), lambda i: (i, 0)),
        pl.BlockSpec((BLK, 1), lambda i: (i, 0)),
        pl.BlockSpec((BLK, 1), lambda i: (i, 0)),
        pl.BlockSpec((BLK, 1), lambda i: (i, 0)),
        pl.BlockSpec((1, 1), lambda i: (0, 0)),
    ],
    out_specs=pl.BlockSpec((BLK, 1), lambda i: (i, 0)),
    out_shape=jax.ShapeDtypeStruct((NP, 1), jnp.float32),
)


def kernel(x, edge_index, W1, b1, W2, b2):
    e = edge_index.shape[1]
    src = edge_index[0].astype(jnp.int32)
    dst = edge_index[1].astype(jnp.int32)
    src_p = jnp.concatenate([src, jnp.zeros((EP - e,), jnp.int32)])
    pad_dst = TRASH + jax.lax.rem(
        jnp.arange(EP - e, dtype=jnp.int32), jnp.int32(NP - N)
    )
    dst_p = jnp.concatenate([dst, pad_dst])
    src2d = src_p.reshape(NCH, CH)
    dst2d = dst_p.reshape(NCH, CH)
    zeros_np = jnp.zeros((NP,), jnp.float32)
    x_p = jnp.concatenate(
        [x.astype(jnp.float32), jnp.zeros((NP - N, D), jnp.float32)]
    )

    degp = _deg_kernel(dst2d, zeros_np)
    deg0 = degp[0].reshape(NP, 1)
    deg1 = degp[1].reshape(NP, 1)
    u, dinv = _mm1(x_p, W1, deg0, deg1)
    acc = _agg_kernel(src2d, dst2d, u)
    v = _mm2(acc, dinv, W2, b1.reshape(1, H))
    aggp = _agg2_kernel(src2d, dst2d, v.reshape(NP), zeros_np)
    out = _fin(
        aggp[0].reshape(NP, 1), aggp[1].reshape(NP, 1), v, dinv,
        b2.reshape(1, 1),
    )
    return out[:N]


# speedup vs baseline: 1.9073x; 1.8952x over previous
import functools

import jax
import jax.numpy as jnp
from jax import lax
from jax.experimental import pallas as pl
from jax.experimental.pallas import tpu as pltpu
from jax.experimental.pallas import tpu_sc as plsc

N = 10000
D = 128
H = 128
NP = 10240
TRASH = 10000
NC = 2
NS = 16
L = 16
NW = NC * NS
CH = 128
CPT = 80
EP = NW * CPT * CH
NCH = EP // CH
BLK = 1024
TG = NP // BLK
NPS = NP // NS

_mesh = plsc.VectorSubcoreMesh(
    core_axis_name="c", subcore_axis_name="s", num_cores=NC, num_subcores=NS
)


@functools.partial(
    pl.kernel,
    out_type=jax.ShapeDtypeStruct((NC, NP), jnp.float32),
    mesh=_mesh,
    scratch_types=[
        pltpu.VMEM((CPT, CH), jnp.int32),
        pltpu.VMEM((CH,), jnp.float32),
        pltpu.VMEM_SHARED((NP,), jnp.float32),
    ],
)
def _deg_kernel(dst2d, zeros_np, deg_out, didx, ones, deg_sp):
    cid = lax.axis_index("c")
    sid = lax.axis_index("s")
    wid = cid * NS + sid
    for j in range(CH // L):
        ones[pl.ds(j * L, L)] = jnp.ones((L,), jnp.float32)
    pltpu.sync_copy(dst2d.at[pl.ds(wid * CPT, CPT)], didx)

    @pl.when(sid == 0)
    def _():
        pltpu.sync_copy(zeros_np, deg_sp)

    plsc.subcore_barrier()

    def body(k, carry):
        pltpu.sync_copy(ones, deg_sp.at[didx.at[k]], add=True)
        return carry

    lax.fori_loop(0, CPT, body, 0)
    plsc.subcore_barrier()
    pltpu.sync_copy(
        deg_sp.at[pl.ds(sid * NPS, NPS)],
        deg_out.at[cid, pl.ds(sid * NPS, NPS)],
    )


GRP = 16
NGA = NCH // NS // GRP


@functools.partial(
    pl.kernel,
    out_type=jax.ShapeDtypeStruct((NP, H), jnp.float32),
    mesh=_mesh,
    scratch_types=[
        pltpu.VMEM((GRP, CH), jnp.int32),
        pltpu.VMEM((GRP, CH), jnp.int32),
        pltpu.VMEM((2, CH, H), jnp.float32),
        pltpu.VMEM_SHARED((NP, H), jnp.float32),
        pltpu.SemaphoreType.DMA,
        pltpu.SemaphoreType.DMA,
    ],
)
def _agg_kernel(src2d, dst2d, u_hbm, acc_out,
                sidx, didx, rows, acc_sp, sem0, sem1):
    cid = lax.axis_index("c")
    sid = lax.axis_index("s")
    ng = jnp.where(cid == 0, NGA, 0)
    start = sid * (NGA * GRP)

    @pl.when((cid == 0) & (sid == 0))
    def _():
        pltpu.sync_copy(u_hbm, acc_sp)

    plsc.subcore_barrier()

    def gbody(g, carry):
        base = start + g * GRP
        pltpu.sync_copy(src2d.at[pl.ds(base, GRP)], sidx)
        pltpu.sync_copy(dst2d.at[pl.ds(base, GRP)], didx)
        pltpu.async_copy(u_hbm.at[sidx.at[0]], rows.at[0], sem0)

        def body(i, c2):
            k0 = i * 2
            pltpu.async_copy(u_hbm.at[sidx.at[k0 + 1]], rows.at[1], sem1)
            pltpu.make_async_copy(
                u_hbm.at[sidx.at[k0]], rows.at[0], sem0).wait()
            pltpu.sync_copy(rows.at[0], acc_sp.at[didx.at[k0]], add=True)

            @pl.when(k0 + 2 < GRP)
            def _():
                pltpu.async_copy(u_hbm.at[sidx.at[k0 + 2]], rows.at[0], sem0)

            pltpu.make_async_copy(
                u_hbm.at[sidx.at[k0 + 1]], rows.at[1], sem1).wait()
            pltpu.sync_copy(rows.at[1], acc_sp.at[didx.at[k0 + 1]], add=True)
            return c2

        lax.fori_loop(0, GRP // 2, body, 0)
        return carry

    lax.fori_loop(0, ng, gbody, 0)
    plsc.subcore_barrier()

    @pl.when(cid == 0)
    def _():
        pltpu.sync_copy(
            acc_sp.at[pl.ds(sid * NPS, NPS)],
            acc_out.at[pl.ds(sid * NPS, NPS)],
        )


@functools.partial(
    pl.kernel,
    out_type=jax.ShapeDtypeStruct((NC, NP), jnp.float32),
    mesh=_mesh,
    scratch_types=[
        pltpu.VMEM((CPT, CH), jnp.int32),
        pltpu.VMEM((CPT, CH), jnp.int32),
        pltpu.VMEM((2, CH), jnp.float32),
        pltpu.VMEM_SHARED((NP,), jnp.float32),
        pltpu.SemaphoreType.DMA,
        pltpu.SemaphoreType.DMA,
    ],
)
def _agg2_kernel(src2d, dst2d, v_hbm, zeros_np, agg_out,
                 sidx, didx, vals, agg_sp, sem0, sem1):
    cid = lax.axis_index("c")
    sid = lax.axis_index("s")
    wid = cid * NS + sid
    pltpu.sync_copy(src2d.at[pl.ds(wid * CPT, CPT)], sidx)
    pltpu.sync_copy(dst2d.at[pl.ds(wid * CPT, CPT)], didx)

    @pl.when(sid == 0)
    def _():
        pltpu.sync_copy(zeros_np, agg_sp)

    plsc.subcore_barrier()
    pltpu.async_copy(v_hbm.at[sidx.at[0]], vals.at[0], sem0)

    def body(i, carry):
        k0 = i * 2
        pltpu.async_copy(v_hbm.at[sidx.at[k0 + 1]], vals.at[1], sem1)
        pltpu.make_async_copy(
            v_hbm.at[sidx.at[k0]], vals.at[0], sem0).wait()
        pltpu.sync_copy(vals.at[0], agg_sp.at[didx.at[k0]], add=True)

        @pl.when(k0 + 2 < CPT)
        def _():
            pltpu.async_copy(v_hbm.at[sidx.at[k0 + 2]], vals.at[0], sem0)

        pltpu.make_async_copy(
            v_hbm.at[sidx.at[k0 + 1]], vals.at[1], sem1).wait()
        pltpu.sync_copy(vals.at[1], agg_sp.at[didx.at[k0 + 1]], add=True)
        return carry

    lax.fori_loop(0, CPT // 2, body, 0)
    plsc.subcore_barrier()
    pltpu.sync_copy(
        agg_sp.at[pl.ds(sid * NPS, NPS)],
        agg_out.at[cid, pl.ds(sid * NPS, NPS)],
    )


def _mm1_body(x_ref, w_ref, d0_ref, d1_ref, u_ref, dinv_ref):
    h = jnp.dot(x_ref[...], w_ref[...], preferred_element_type=jnp.float32)
    deg = d0_ref[...] + d1_ref[...] + 1.0
    dinv = lax.rsqrt(deg)
    u_ref[...] = h * dinv
    dinv_ref[...] = dinv


_mm1 = pl.pallas_call(
    _mm1_body,
    grid=(TG,),
    in_specs=[
        pl.BlockSpec((BLK, D), lambda i: (i, 0)),
        pl.BlockSpec((D, H), lambda i: (0, 0)),
        pl.BlockSpec((BLK, 1), lambda i: (i, 0)),
        pl.BlockSpec((BLK, 1), lambda i: (i, 0)),
    ],
    out_specs=[
        pl.BlockSpec((BLK, H), lambda i: (i, 0)),
        pl.BlockSpec((BLK, 1), lambda i: (i, 0)),
    ],
    out_shape=[
        jax.ShapeDtypeStruct((NP, H), jnp.float32),
        jax.ShapeDtypeStruct((NP, 1), jnp.float32),
    ],
)


def _mm2_body(a0_ref, dinv_ref, w2_ref, b1_ref, v_ref):
    dinv = dinv_ref[...]
    out1 = dinv * a0_ref[...] + b1_ref[...]
    a = jnp.maximum(out1, 0.0)
    z = jnp.dot(a, w2_ref[...], preferred_element_type=jnp.float32)
    v_ref[...] = z * dinv


_mm2 = pl.pallas_call(
    _mm2_body,
    grid=(TG,),
    in_specs=[
        pl.BlockSpec((BLK, H), lambda i: (i, 0)),
        pl.BlockSpec((BLK, 1), lambda i: (i, 0)),
        pl.BlockSpec((H, 1), lambda i: (0, 0)),
        pl.BlockSpec((1, H), lambda i: (0, 0)),
    ],
    out_specs=pl.BlockSpec((BLK, 1), lambda i: (i, 0)),
    out_shape=jax.ShapeDtypeStruct((NP, 1), jnp.float32),
)


def _fin_body(a0_ref, a1_ref, v_ref, dinv_ref, b2_ref, o_ref):
    o_ref[...] = (
        dinv_ref[...] * (a0_ref[...] + a1_ref[...] + v_ref[...]) + b2_ref[...]
    )


_fin = pl.pallas_call(
    _fin_body,
    grid=(TG,),
    in_specs=[
        pl.BlockSpec((BLK, 1), lambda i: (i, 0)),
        pl.BlockSpec((BLK, 1), lambda i: (i, 0)),
        pl.BlockSpec((BLK, 1), lambda i: (i, 0)),
        pl.BlockSpec((BLK, 1), lambda i: (i, 0)),
        pl.BlockSpec((1, 1), lambda i: (0, 0)),
    ],
    out_specs=pl.BlockSpec((BLK, 1), lambda i: (i, 0)),
    out_shape=jax.ShapeDtypeStruct((NP, 1), jnp.float32),
)


def kernel(x, edge_index, W1, b1, W2, b2):
    e = edge_index.shape[1]
    src = edge_index[0].astype(jnp.int32)
    dst = edge_index[1].astype(jnp.int32)
    pad_i = jnp.arange(EP - e, dtype=jnp.int32)
    pad_src = jax.lax.rem(pad_i, jnp.int32(N))
    pad_dst = TRASH + jax.lax.rem(pad_i, jnp.int32(NP - N))
    src_p = jnp.concatenate([src, pad_src])
    dst_p = jnp.concatenate([dst, pad_dst])
    src2d = src_p.reshape(NCH, CH)
    dst2d = dst_p.reshape(NCH, CH)
    zeros_np = jnp.zeros((NP,), jnp.float32)
    x_p = jnp.concatenate(
        [x.astype(jnp.float32), jnp.zeros((NP - N, D), jnp.float32)]
    )

    degp = _deg_kernel(dst2d, zeros_np)
    deg0 = degp[0].reshape(NP, 1)
    deg1 = degp[1].reshape(NP, 1)
    u, dinv = _mm1(x_p, W1, deg0, deg1)
    acc = _agg_kernel(src2d, dst2d, u)
    v = _mm2(acc, dinv, W2, b1.reshape(1, H))
    aggp = _agg2_kernel(src2d, dst2d, v.reshape(NP), zeros_np)
    out = _fin(
        aggp[0].reshape(NP, 1), aggp[1].reshape(NP, 1), v, dinv,
        b2.reshape(1, 1),
    )
    return out[:N]


# speedup vs baseline: 2.5134x; 1.3178x over previous
import functools

import jax
import jax.numpy as jnp
from jax import lax
from jax.experimental import pallas as pl
from jax.experimental.pallas import tpu as pltpu
from jax.experimental.pallas import tpu_sc as plsc

N = 10000
D = 128
H = 128
NP = 10240
TRASH = 10000
NC = 2
NS = 16
L = 16
NW = NC * NS
CH = 128
CPT = 80
EP = NW * CPT * CH
NCH = EP // CH
BLK = 1024
TG = NP // BLK
NPS = NP // NS

_mesh = plsc.VectorSubcoreMesh(
    core_axis_name="c", subcore_axis_name="s", num_cores=NC, num_subcores=NS
)


@functools.partial(
    pl.kernel,
    out_type=jax.ShapeDtypeStruct((NC, NP), jnp.float32),
    mesh=_mesh,
    scratch_types=[
        pltpu.VMEM((CPT, CH), jnp.int32),
        pltpu.VMEM((CH,), jnp.float32),
        pltpu.VMEM_SHARED((NP,), jnp.float32),
    ],
)
def _deg_kernel(dst2d, zeros_np, deg_out, didx, ones, deg_sp):
    cid = lax.axis_index("c")
    sid = lax.axis_index("s")
    wid = cid * NS + sid
    for j in range(CH // L):
        ones[pl.ds(j * L, L)] = jnp.ones((L,), jnp.float32)
    pltpu.sync_copy(dst2d.at[pl.ds(wid * CPT, CPT)], didx)

    @pl.when(sid == 0)
    def _():
        pltpu.sync_copy(zeros_np, deg_sp)

    plsc.subcore_barrier()

    def body(k, carry):
        pltpu.sync_copy(ones, deg_sp.at[didx.at[k]], add=True)
        return carry

    lax.fori_loop(0, CPT, body, 0)
    plsc.subcore_barrier()
    pltpu.sync_copy(
        deg_sp.at[pl.ds(sid * NPS, NPS)],
        deg_out.at[cid, pl.ds(sid * NPS, NPS)],
    )


GRP = 16
NGA = NCH // NW // GRP


@functools.partial(
    pl.kernel,
    out_type=jax.ShapeDtypeStruct((NC, NP, H), jnp.float32),
    mesh=_mesh,
    scratch_types=[
        pltpu.VMEM((GRP, CH), jnp.int32),
        pltpu.VMEM((GRP, CH), jnp.int32),
        pltpu.VMEM((2, CH, H), jnp.float32),
        pltpu.VMEM_SHARED((NP, H), jnp.float32),
        pltpu.SemaphoreType.DMA,
        pltpu.SemaphoreType.DMA,
    ],
)
def _agg_kernel(src2d, dst2d, u_hbm, acc_out,
                sidx, didx, rows, acc_sp, sem0, sem1):
    cid = lax.axis_index("c")
    sid = lax.axis_index("s")
    wid = cid * NS + sid
    start = wid * (NGA * GRP)

    @pl.when(sid == 0)
    def _():
        pltpu.sync_copy(u_hbm, acc_sp)

    plsc.subcore_barrier()

    def gbody(g, carry):
        base = start + g * GRP
        pltpu.sync_copy(src2d.at[pl.ds(base, GRP)], sidx)
        pltpu.sync_copy(dst2d.at[pl.ds(base, GRP)], didx)
        pltpu.async_copy(u_hbm.at[sidx.at[0]], rows.at[0], sem0)

        def body(i, c2):
            k0 = i * 2
            pltpu.async_copy(u_hbm.at[sidx.at[k0 + 1]], rows.at[1], sem1)
            pltpu.make_async_copy(
                u_hbm.at[sidx.at[k0]], rows.at[0], sem0).wait()
            pltpu.sync_copy(rows.at[0], acc_sp.at[didx.at[k0]], add=True)

            @pl.when(k0 + 2 < GRP)
            def _():
                pltpu.async_copy(u_hbm.at[sidx.at[k0 + 2]], rows.at[0], sem0)

            pltpu.make_async_copy(
                u_hbm.at[sidx.at[k0 + 1]], rows.at[1], sem1).wait()
            pltpu.sync_copy(rows.at[1], acc_sp.at[didx.at[k0 + 1]], add=True)
            return c2

        lax.fori_loop(0, GRP // 2, body, 0)
        return carry

    lax.fori_loop(0, NGA, gbody, 0)
    plsc.subcore_barrier()
    pltpu.sync_copy(
        acc_sp.at[pl.ds(sid * NPS, NPS)],
        acc_out.at[cid, pl.ds(sid * NPS, NPS)],
    )


@functools.partial(
    pl.kernel,
    out_type=jax.ShapeDtypeStruct((NC, NP), jnp.float32),
    mesh=_mesh,
    scratch_types=[
        pltpu.VMEM((CPT, CH), jnp.int32),
        pltpu.VMEM((CPT, CH), jnp.int32),
        pltpu.VMEM((2, CH), jnp.float32),
        pltpu.VMEM_SHARED((NP,), jnp.float32),
        pltpu.SemaphoreType.DMA,
        pltpu.SemaphoreType.DMA,
    ],
)
def _agg2_kernel(src2d, dst2d, v_hbm, zeros_np, agg_out,
                 sidx, didx, vals, agg_sp, sem0, sem1):
    cid = lax.axis_index("c")
    sid = lax.axis_index("s")
    wid = cid * NS + sid
    pltpu.sync_copy(src2d.at[pl.ds(wid * CPT, CPT)], sidx)
    pltpu.sync_copy(dst2d.at[pl.ds(wid * CPT, CPT)], didx)

    @pl.when(sid == 0)
    def _():
        pltpu.sync_copy(zeros_np, agg_sp)

    plsc.subcore_barrier()
    pltpu.async_copy(v_hbm.at[sidx.at[0]], vals.at[0], sem0)

    def body(i, carry):
        k0 = i * 2
        pltpu.async_copy(v_hbm.at[sidx.at[k0 + 1]], vals.at[1], sem1)
        pltpu.make_async_copy(
            v_hbm.at[sidx.at[k0]], vals.at[0], sem0).wait()
        pltpu.sync_copy(vals.at[0], agg_sp.at[didx.at[k0]], add=True)

        @pl.when(k0 + 2 < CPT)
        def _():
            pltpu.async_copy(v_hbm.at[sidx.at[k0 + 2]], vals.at[0], sem0)

        pltpu.make_async_copy(
            v_hbm.at[sidx.at[k0 + 1]], vals.at[1], sem1).wait()
        pltpu.sync_copy(vals.at[1], agg_sp.at[didx.at[k0 + 1]], add=True)
        return carry

    lax.fori_loop(0, CPT // 2, body, 0)
    plsc.subcore_barrier()
    pltpu.sync_copy(
        agg_sp.at[pl.ds(sid * NPS, NPS)],
        agg_out.at[cid, pl.ds(sid * NPS, NPS)],
    )


def _mm1_body(x_ref, w_ref, d0_ref, d1_ref, u_ref, dinv_ref):
    h = jnp.dot(x_ref[...], w_ref[...], preferred_element_type=jnp.float32)
    deg = d0_ref[...] + d1_ref[...] + 1.0
    dinv = lax.rsqrt(deg)
    u_ref[...] = h * dinv
    dinv_ref[...] = dinv


_mm1 = pl.pallas_call(
    _mm1_body,
    grid=(TG,),
    in_specs=[
        pl.BlockSpec((BLK, D), lambda i: (i, 0)),
        pl.BlockSpec((D, H), lambda i: (0, 0)),
        pl.BlockSpec((BLK, 1), lambda i: (i, 0)),
        pl.BlockSpec((BLK, 1), lambda i: (i, 0)),
    ],
    out_specs=[
        pl.BlockSpec((BLK, H), lambda i: (i, 0)),
        pl.BlockSpec((BLK, 1), lambda i: (i, 0)),
    ],
    out_shape=[
        jax.ShapeDtypeStruct((NP, H), jnp.float32),
        jax.ShapeDtypeStruct((NP, 1), jnp.float32),
    ],
)


def _mm2_body(a0_ref, a1_ref, u_ref, dinv_ref, w2_ref, b1_ref, v_ref):
    dinv = dinv_ref[...]
    out1 = dinv * (a0_ref[...] + a1_ref[...] - u_ref[...]) + b1_ref[...]
    a = jnp.maximum(out1, 0.0)
    z = jnp.dot(a, w2_ref[...], preferred_element_type=jnp.float32)
    v_ref[...] = z * dinv


_mm2 = pl.pallas_call(
    _mm2_body,
    grid=(TG,),
    in_specs=[
        pl.BlockSpec((BLK, H), lambda i: (i, 0)),
        pl.BlockSpec((BLK, H), lambda i: (i, 0)),
        pl.BlockSpec((BLK, H), lambda i: (i, 0)),
        pl.BlockSpec((BLK, 1), lambda i: (i, 0)),
        pl.BlockSpec((H, 1), lambda i: (0, 0)),
        pl.BlockSpec((1, H), lambda i: (0, 0)),
    ],
    out_specs=pl.BlockSpec((BLK, 1), lambda i: (i, 0)),
    out_shape=jax.ShapeDtypeStruct((NP, 1), jnp.float32),
)


def _fin_body(a0_ref, a1_ref, v_ref, dinv_ref, b2_ref, o_ref):
    o_ref[...] = (
        dinv_ref[...] * (a0_ref[...] + a1_ref[...] + v_ref[...]) + b2_ref[...]
    )


_fin = pl.pallas_call(
    _fin_body,
    grid=(TG,),
    in_specs=[
        pl.BlockSpec((BLK, 1), lambda i: (i, 0)),
        pl.BlockSpec((BLK, 1), lambda i: (i, 0)),
        pl.BlockSpec((BLK, 1), lambda i: (i, 0)),
        pl.BlockSpec((BLK, 1), lambda i: (i, 0)),
        pl.BlockSpec((1, 1), lambda i: (0, 0)),
    ],
    out_specs=pl.BlockSpec((BLK, 1), lambda i: (i, 0)),
    out_shape=jax.ShapeDtypeStruct((NP, 1), jnp.float32),
)


def kernel(x, edge_index, W1, b1, W2, b2):
    e = edge_index.shape[1]
    src = edge_index[0].astype(jnp.int32)
    dst = edge_index[1].astype(jnp.int32)
    pad_i = jnp.arange(EP - e, dtype=jnp.int32)
    pad_src = jax.lax.rem(pad_i, jnp.int32(N))
    pad_dst = TRASH + jax.lax.rem(pad_i, jnp.int32(NP - N))
    src_p = jnp.concatenate([src, pad_src])
    dst_p = jnp.concatenate([dst, pad_dst])
    src2d = src_p.reshape(NCH, CH)
    dst2d = dst_p.reshape(NCH, CH)
    zeros_np = jnp.zeros((NP,), jnp.float32)
    x_p = jnp.concatenate(
        [x.astype(jnp.float32), jnp.zeros((NP - N, D), jnp.float32)]
    )

    degp = _deg_kernel(dst2d, zeros_np)
    deg0 = degp[0].reshape(NP, 1)
    deg1 = degp[1].reshape(NP, 1)
    u, dinv = _mm1(x_p, W1, deg0, deg1)
    accp = _agg_kernel(src2d, dst2d, u)
    v = _mm2(accp[0], accp[1], u, dinv, W2, b1.reshape(1, H))
    aggp = _agg2_kernel(src2d, dst2d, v.reshape(NP), zeros_np)
    out = _fin(
        aggp[0].reshape(NP, 1), aggp[1].reshape(NP, 1), v, dinv,
        b2.reshape(1, 1),
    )
    return out[:N]


# speedup vs baseline: 2.5839x; 1.0280x over previous
import functools

import jax
import jax.numpy as jnp
from jax import lax
from jax.experimental import pallas as pl
from jax.experimental.pallas import tpu as pltpu
from jax.experimental.pallas import tpu_sc as plsc

N = 10000
D = 128
H = 128
NP = 10240
TRASH = 10000
NC = 2
NS = 16
L = 16
NW = NC * NS
CH = 128
CPT = 80
EP = NW * CPT * CH
NCH = EP // CH
BLK = 1024
TG = NP // BLK
NPS = NP // NS

_mesh = plsc.VectorSubcoreMesh(
    core_axis_name="c", subcore_axis_name="s", num_cores=NC, num_subcores=NS
)


@functools.partial(
    pl.kernel,
    out_type=jax.ShapeDtypeStruct((NC, NP), jnp.float32),
    mesh=_mesh,
    scratch_types=[
        pltpu.VMEM((CPT, CH), jnp.int32),
        pltpu.VMEM((CH,), jnp.float32),
        pltpu.VMEM_SHARED((NP,), jnp.float32),
    ],
)
def _deg_kernel(dst2d, zeros_np, deg_out, didx, ones, deg_sp):
    cid = lax.axis_index("c")
    sid = lax.axis_index("s")
    wid = cid * NS + sid
    for j in range(CH // L):
        ones[pl.ds(j * L, L)] = jnp.ones((L,), jnp.float32)
    pltpu.sync_copy(dst2d.at[pl.ds(wid * CPT, CPT)], didx)

    @pl.when(sid == 0)
    def _():
        pltpu.sync_copy(zeros_np, deg_sp)

    plsc.subcore_barrier()

    def body(k, carry):
        pltpu.sync_copy(ones, deg_sp.at[didx.at[k]], add=True)
        return carry

    lax.fori_loop(0, CPT, body, 0)
    plsc.subcore_barrier()
    pltpu.sync_copy(
        deg_sp.at[pl.ds(sid * NPS, NPS)],
        deg_out.at[cid, pl.ds(sid * NPS, NPS)],
    )


GRP = 16
NGA = NCH // NW // GRP


@functools.partial(
    pl.kernel,
    out_type=jax.ShapeDtypeStruct((NC, NP, H), jnp.float32),
    mesh=_mesh,
    scratch_types=[
        pltpu.VMEM((2, GRP, CH), jnp.int32),
        pltpu.VMEM((2, GRP, CH), jnp.int32),
        pltpu.VMEM((2, CH, H), jnp.float32),
        pltpu.VMEM_SHARED((NP, H), jnp.float32),
        pltpu.SemaphoreType.DMA,
        pltpu.SemaphoreType.DMA,
        pltpu.SemaphoreType.DMA,
    ],
)
def _agg_kernel(src2d, dst2d, u_hbm, acc_out,
                sidx, didx, rows, acc_sp, sem0, sem1, isem):
    cid = lax.axis_index("c")
    sid = lax.axis_index("s")
    wid = cid * NS + sid
    start = wid * (NGA * GRP)

    pltpu.sync_copy(src2d.at[pl.ds(start, GRP)], sidx.at[0])
    pltpu.sync_copy(dst2d.at[pl.ds(start, GRP)], didx.at[0])

    @pl.when(sid == 0)
    def _():
        pltpu.sync_copy(u_hbm, acc_sp)

    plsc.subcore_barrier()

    def gbody(g, carry):
        p = lax.rem(g, 2)
        pnx = lax.rem(g + 1, 2)
        nbase = start + (g + 1) * GRP

        @pl.when(g + 1 < NGA)
        def _():
            pltpu.async_copy(src2d.at[pl.ds(nbase, GRP)], sidx.at[pnx], isem)
            pltpu.async_copy(dst2d.at[pl.ds(nbase, GRP)], didx.at[pnx], isem)

        sx = sidx.at[p]
        dx = didx.at[p]
        pltpu.async_copy(u_hbm.at[sx.at[0]], rows.at[0], sem0)

        def body(i, c2):
            k0 = i * 2
            pltpu.async_copy(u_hbm.at[sx.at[k0 + 1]], rows.at[1], sem1)
            pltpu.make_async_copy(
                u_hbm.at[sx.at[k0]], rows.at[0], sem0).wait()
            pltpu.sync_copy(rows.at[0], acc_sp.at[dx.at[k0]], add=True)

            @pl.when(k0 + 2 < GRP)
            def _():
                pltpu.async_copy(u_hbm.at[sx.at[k0 + 2]], rows.at[0], sem0)

            pltpu.make_async_copy(
                u_hbm.at[sx.at[k0 + 1]], rows.at[1], sem1).wait()
            pltpu.sync_copy(rows.at[1], acc_sp.at[dx.at[k0 + 1]], add=True)
            return c2

        lax.fori_loop(0, GRP // 2, body, 0)

        @pl.when(g + 1 < NGA)
        def _():
            pltpu.make_async_copy(
                src2d.at[pl.ds(nbase, GRP)], sidx.at[pnx], isem).wait()
            pltpu.make_async_copy(
                dst2d.at[pl.ds(nbase, GRP)], didx.at[pnx], isem).wait()

        return carry

    lax.fori_loop(0, NGA, gbody, 0)
    plsc.subcore_barrier()
    pltpu.sync_copy(
        acc_sp.at[pl.ds(sid * NPS, NPS)],
        acc_out.at[cid, pl.ds(sid * NPS, NPS)],
    )


@functools.partial(
    pl.kernel,
    out_type=jax.ShapeDtypeStruct((NC, NP), jnp.float32),
    mesh=_mesh,
    scratch_types=[
        pltpu.VMEM((CPT, CH), jnp.int32),
        pltpu.VMEM((CPT, CH), jnp.int32),
        pltpu.VMEM((8, CH), jnp.float32),
        pltpu.VMEM_SHARED((NP,), jnp.float32),
        pltpu.SemaphoreType.DMA((8,)),
        pltpu.SemaphoreType.DMA((8,)),
    ],
)
def _agg2_kernel(src2d, dst2d, v_hbm, zeros_np, agg_out,
                 sidx, didx, vals, agg_sp, gsem, ssem):
    cid = lax.axis_index("c")
    sid = lax.axis_index("s")
    wid = cid * NS + sid
    pltpu.sync_copy(src2d.at[pl.ds(wid * CPT, CPT)], sidx)
    pltpu.sync_copy(dst2d.at[pl.ds(wid * CPT, CPT)], didx)

    @pl.when(sid == 0)
    def _():
        pltpu.sync_copy(zeros_np, agg_sp)

    plsc.subcore_barrier()

    def wave(w, carry):
        k0 = w * 8
        for b in range(8):
            pltpu.async_copy(
                v_hbm.at[sidx.at[k0 + b]], vals.at[b], gsem.at[b])
        for b in range(8):
            pltpu.make_async_copy(
                v_hbm.at[sidx.at[k0 + b]], vals.at[b], gsem.at[b]).wait()
            pltpu.async_copy(
                vals.at[b], agg_sp.at[didx.at[k0 + b]], ssem.at[b],
                add=True)
        for b in range(8):
            pltpu.make_async_copy(
                vals.at[b], agg_sp.at[didx.at[k0 + b]], ssem.at[b]).wait()
        return carry

    lax.fori_loop(0, CPT // 8, wave, 0)
    plsc.subcore_barrier()
    pltpu.sync_copy(
        agg_sp.at[pl.ds(sid * NPS, NPS)],
        agg_out.at[cid, pl.ds(sid * NPS, NPS)],
    )


def _mm1_body(x_ref, w_ref, d0_ref, d1_ref, u_ref, dinv_ref):
    h = jnp.dot(x_ref[...], w_ref[...], preferred_element_type=jnp.float32)
    deg = d0_ref[...] + d1_ref[...] + 1.0
    dinv = lax.rsqrt(deg)
    u_ref[...] = h * dinv
    dinv_ref[...] = dinv


_mm1 = pl.pallas_call(
    _mm1_body,
    grid=(TG,),
    in_specs=[
        pl.BlockSpec((BLK, D), lambda i: (i, 0)),
        pl.BlockSpec((D, H), lambda i: (0, 0)),
        pl.BlockSpec((BLK, 1), lambda i: (i, 0)),
        pl.BlockSpec((BLK, 1), lambda i: (i, 0)),
    ],
    out_specs=[
        pl.BlockSpec((BLK, H), lambda i: (i, 0)),
        pl.BlockSpec((BLK, 1), lambda i: (i, 0)),
    ],
    out_shape=[
        jax.ShapeDtypeStruct((NP, H), jnp.float32),
        jax.ShapeDtypeStruct((NP, 1), jnp.float32),
    ],
)


def _mm2_body(a0_ref, a1_ref, u_ref, dinv_ref, w2_ref, b1_ref, v_ref):
    dinv = dinv_ref[...]
    out1 = dinv * (a0_ref[...] + a1_ref[...] - u_ref[...]) + b1_ref[...]
    a = jnp.maximum(out1, 0.0)
    z = jnp.dot(a, w2_ref[...], preferred_element_type=jnp.float32)
    v_ref[...] = z * dinv


_mm2 = pl.pallas_call(
    _mm2_body,
    grid=(TG,),
    in_specs=[
        pl.BlockSpec((BLK, H), lambda i: (i, 0)),
        pl.BlockSpec((BLK, H), lambda i: (i, 0)),
        pl.BlockSpec((BLK, H), lambda i: (i, 0)),
        pl.BlockSpec((BLK, 1), lambda i: (i, 0)),
        pl.BlockSpec((H, 1), lambda i: (0, 0)),
        pl.BlockSpec((1, H), lambda i: (0, 0)),
    ],
    out_specs=pl.BlockSpec((BLK, 1), lambda i: (i, 0)),
    out_shape=jax.ShapeDtypeStruct((NP, 1), jnp.float32),
)


def _fin_body(a0_ref, a1_ref, v_ref, dinv_ref, b2_ref, o_ref):
    o_ref[...] = (
        dinv_ref[...] * (a0_ref[...] + a1_ref[...] + v_ref[...]) + b2_ref[...]
    )


_fin = pl.pallas_call(
    _fin_body,
    grid=(TG,),
    in_specs=[
        pl.BlockSpec((BLK, 1), lambda i: (i, 0)),
        pl.BlockSpec((BLK, 1), lambda i: (i, 0)),
        pl.BlockSpec((BLK, 1), lambda i: (i, 0)),
        pl.BlockSpec((BLK, 1), lambda i: (i, 0)),
        pl.BlockSpec((1, 1), lambda i: (0, 0)),
    ],
    out_specs=pl.BlockSpec((BLK, 1), lambda i: (i, 0)),
    out_shape=jax.ShapeDtypeStruct((NP, 1), jnp.float32),
)


def kernel(x, edge_index, W1, b1, W2, b2):
    e = edge_index.shape[1]
    src = edge_index[0].astype(jnp.int32)
    dst = edge_index[1].astype(jnp.int32)
    pad_i = jnp.arange(EP - e, dtype=jnp.int32)
    pad_src = jax.lax.rem(pad_i, jnp.int32(N))
    pad_dst = TRASH + jax.lax.rem(pad_i, jnp.int32(NP - N))
    src_p = jnp.concatenate([src, pad_src])
    dst_p = jnp.concatenate([dst, pad_dst])
    src2d = src_p.reshape(NCH, CH)
    dst2d = dst_p.reshape(NCH, CH)
    zeros_np = jnp.zeros((NP,), jnp.float32)
    x_p = jnp.concatenate(
        [x.astype(jnp.float32), jnp.zeros((NP - N, D), jnp.float32)]
    )

    degp = _deg_kernel(dst2d, zeros_np)
    deg0 = degp[0].reshape(NP, 1)
    deg1 = degp[1].reshape(NP, 1)
    u, dinv = _mm1(x_p, W1, deg0, deg1)
    accp = _agg_kernel(src2d, dst2d, u)
    v = _mm2(accp[0], accp[1], u, dinv, W2, b1.reshape(1, H))
    aggp = _agg2_kernel(src2d, dst2d, v.reshape(NP), zeros_np)
    out = _fin(
        aggp[0].reshape(NP, 1), aggp[1].reshape(NP, 1), v, dinv,
        b2.reshape(1, 1),
    )
    return out[:N]


# speedup vs baseline: 2.8227x; 1.0925x over previous
import functools

import jax
import jax.numpy as jnp
from jax import lax
from jax.experimental import pallas as pl
from jax.experimental.pallas import tpu as pltpu
from jax.experimental.pallas import tpu_sc as plsc

N = 10000
D = 128
H = 128
NP = 10240
TRASH = 10000
NC = 2
NS = 16
L = 16
NW = NC * NS
CH = 128
CPT = 80
EP = NW * CPT * CH
NCH = EP // CH
BLKN = 1024
TGN = NP // BLKN
NPS = NP // NS

_mesh = plsc.VectorSubcoreMesh(
    core_axis_name="c", subcore_axis_name="s", num_cores=NC, num_subcores=NS
)


@functools.partial(
    pl.kernel,
    out_type=jax.ShapeDtypeStruct((NC, NP), jnp.float32),
    mesh=_mesh,
    scratch_types=[
        pltpu.VMEM((CPT, CH), jnp.int32),
        pltpu.VMEM((CH,), jnp.float32),
        pltpu.VMEM_SHARED((NP,), jnp.float32),
    ],
)
def _deg_kernel(dst2d, zeros_np, deg_out, didx, ones, deg_sp):
    cid = lax.axis_index("c")
    sid = lax.axis_index("s")
    wid = cid * NS + sid
    for j in range(CH // L):
        ones[pl.ds(j * L, L)] = jnp.ones((L,), jnp.float32)
    pltpu.sync_copy(dst2d.at[pl.ds(wid * CPT, CPT)], didx)

    @pl.when(sid == 0)
    def _():
        pltpu.sync_copy(zeros_np, deg_sp)

    plsc.subcore_barrier()

    def body(k, carry):
        pltpu.sync_copy(ones, deg_sp.at[didx.at[k]], add=True)
        return carry

    lax.fori_loop(0, CPT, body, 0)
    plsc.subcore_barrier()
    pltpu.sync_copy(
        deg_sp.at[pl.ds(sid * NPS, NPS)],
        deg_out.at[cid, pl.ds(sid * NPS, NPS)],
    )


GRP = 16
NGA = NCH // NW // GRP


@functools.partial(
    pl.kernel,
    out_type=jax.ShapeDtypeStruct((NC, NP, H), jnp.float32),
    mesh=_mesh,
    scratch_types=[
        pltpu.VMEM((2, GRP, CH), jnp.int32),
        pltpu.VMEM((2, GRP, CH), jnp.int32),
        pltpu.VMEM((2, CH, H), jnp.float32),
        pltpu.VMEM_SHARED((NP, H), jnp.float32),
        pltpu.SemaphoreType.DMA,
        pltpu.SemaphoreType.DMA,
        pltpu.SemaphoreType.DMA,
    ],
)
def _agg_kernel(src2d, dst2d, u_hbm, acc_out,
                sidx, didx, rows, acc_sp, sem0, sem1, isem):
    cid = lax.axis_index("c")
    sid = lax.axis_index("s")
    wid = cid * NS + sid
    start = wid * (NGA * GRP)

    pltpu.sync_copy(src2d.at[pl.ds(start, GRP)], sidx.at[0])
    pltpu.sync_copy(dst2d.at[pl.ds(start, GRP)], didx.at[0])

    @pl.when(sid == 0)
    def _():
        pltpu.sync_copy(u_hbm, acc_sp)

    plsc.subcore_barrier()

    def gbody(g, carry):
        p = lax.rem(g, 2)
        pnx = lax.rem(g + 1, 2)
        nbase = start + (g + 1) * GRP

        @pl.when(g + 1 < NGA)
        def _():
            pltpu.async_copy(src2d.at[pl.ds(nbase, GRP)], sidx.at[pnx], isem)
            pltpu.async_copy(dst2d.at[pl.ds(nbase, GRP)], didx.at[pnx], isem)

        sx = sidx.at[p]
        dx = didx.at[p]
        pltpu.async_copy(u_hbm.at[sx.at[0]], rows.at[0], sem0)

        def body(i, c2):
            k0 = i * 2
            pltpu.async_copy(u_hbm.at[sx.at[k0 + 1]], rows.at[1], sem1)
            pltpu.make_async_copy(
                u_hbm.at[sx.at[k0]], rows.at[0], sem0).wait()
            pltpu.sync_copy(rows.at[0], acc_sp.at[dx.at[k0]], add=True)

            @pl.when(k0 + 2 < GRP)
            def _():
                pltpu.async_copy(u_hbm.at[sx.at[k0 + 2]], rows.at[0], sem0)

            pltpu.make_async_copy(
                u_hbm.at[sx.at[k0 + 1]], rows.at[1], sem1).wait()
            pltpu.sync_copy(rows.at[1], acc_sp.at[dx.at[k0 + 1]], add=True)
            return c2

        lax.fori_loop(0, GRP // 2, body, 0)

        @pl.when(g + 1 < NGA)
        def _():
            pltpu.make_async_copy(
                src2d.at[pl.ds(nbase, GRP)], sidx.at[pnx], isem).wait()
            pltpu.make_async_copy(
                dst2d.at[pl.ds(nbase, GRP)], didx.at[pnx], isem).wait()

        return carry

    lax.fori_loop(0, NGA, gbody, 0)
    plsc.subcore_barrier()
    pltpu.sync_copy(
        acc_sp.at[pl.ds(sid * NPS, NPS)],
        acc_out.at[cid, pl.ds(sid * NPS, NPS)],
    )


@functools.partial(
    pl.kernel,
    out_type=jax.ShapeDtypeStruct((NC, NP), jnp.float32),
    mesh=_mesh,
    scratch_types=[
        pltpu.VMEM((CPT, CH), jnp.int32),
        pltpu.VMEM((CPT, CH), jnp.int32),
        pltpu.VMEM((8, CH), jnp.float32),
        pltpu.VMEM_SHARED((NP,), jnp.float32),
        pltpu.SemaphoreType.DMA((8,)),
        pltpu.SemaphoreType.DMA((8,)),
    ],
)
def _agg2_kernel(src2d, dst2d, v_hbm, zeros_np, agg_out,
                 sidx, didx, vals, agg_sp, gsem, ssem):
    cid = lax.axis_index("c")
    sid = lax.axis_index("s")
    wid = cid * NS + sid
    pltpu.sync_copy(src2d.at[pl.ds(wid * CPT, CPT)], sidx)
    pltpu.sync_copy(dst2d.at[pl.ds(wid * CPT, CPT)], didx)

    @pl.when(sid == 0)
    def _():
        pltpu.sync_copy(zeros_np, agg_sp)

    plsc.subcore_barrier()

    def wave(w, carry):
        k0 = w * 8
        for b in range(8):
            pltpu.async_copy(
                v_hbm.at[sidx.at[k0 + b]], vals.at[b], gsem.at[b])
        for b in range(8):
            pltpu.make_async_copy(
                v_hbm.at[sidx.at[k0 + b]], vals.at[b], gsem.at[b]).wait()
            pltpu.async_copy(
                vals.at[b], agg_sp.at[didx.at[k0 + b]], ssem.at[b],
                add=True)
        for b in range(8):
            pltpu.make_async_copy(
                vals.at[b], agg_sp.at[didx.at[k0 + b]], ssem.at[b]).wait()
        return carry

    lax.fori_loop(0, CPT // 8, wave, 0)
    plsc.subcore_barrier()
    pltpu.sync_copy(
        agg_sp.at[pl.ds(sid * NPS, NPS)],
        agg_out.at[cid, pl.ds(sid * NPS, NPS)],
    )


def _mm1_body(x_ref, w_ref, dg_ref, u_ref, dinv_ref):
    h = jnp.dot(x_ref[...], w_ref[...], preferred_element_type=jnp.float32)
    degrow = dg_ref[0:1, :] + dg_ref[1:2, :] + 1.0
    dinvrow = lax.rsqrt(degrow)
    dinvcol = jnp.transpose(dinvrow, (1, 0))
    u_ref[...] = h * dinvcol
    dinv_ref[...] = jnp.reshape(dinvrow, (BLKN,))


_mm1 = pl.pallas_call(
    _mm1_body,
    grid=(TGN,),
    in_specs=[
        pl.BlockSpec((BLKN, D), lambda i: (i, 0)),
        pl.BlockSpec((D, H), lambda i: (0, 0)),
        pl.BlockSpec((2, BLKN), lambda i: (0, i)),
    ],
    out_specs=[
        pl.BlockSpec((BLKN, H), lambda i: (i, 0)),
        pl.BlockSpec((BLKN,), lambda i: (i,)),
    ],
    out_shape=[
        jax.ShapeDtypeStruct((NP, H), jnp.float32),
        jax.ShapeDtypeStruct((NP,), jnp.float32),
    ],
)


def _mm2_body(a0_ref, a1_ref, u_ref, dinv_ref, w2_ref, b1_ref, v_ref):
    dinvrow = jnp.reshape(dinv_ref[...], (1, BLKN))
    dinvcol = jnp.transpose(dinvrow, (1, 0))
    acc = a0_ref[0] + a1_ref[0] - u_ref[...]
    a = jnp.maximum(dinvcol * acc + b1_ref[...], 0.0)
    zrow = lax.dot_general(
        w2_ref[...], a, (((0,), (1,)), ((), ())),
        preferred_element_type=jnp.float32)
    v_ref[...] = jnp.reshape(zrow * dinvrow, (BLKN,))


_mm2 = pl.pallas_call(
    _mm2_body,
    grid=(TGN,),
    in_specs=[
        pl.BlockSpec((1, BLKN, H), lambda i: (0, i, 0)),
        pl.BlockSpec((1, BLKN, H), lambda i: (1, i, 0)),
        pl.BlockSpec((BLKN, H), lambda i: (i, 0)),
        pl.BlockSpec((BLKN,), lambda i: (i,)),
        pl.BlockSpec((H, 1), lambda i: (0, 0)),
        pl.BlockSpec((1, H), lambda i: (0, 0)),
    ],
    out_specs=pl.BlockSpec((BLKN,), lambda i: (i,)),
    out_shape=jax.ShapeDtypeStruct((NP,), jnp.float32),
)


def _fin_body(ag_ref, v_ref, dinv_ref, b2_ref, o_ref):
    dinvrow = jnp.reshape(dinv_ref[...], (1, BLKN))
    vrow = jnp.reshape(v_ref[...], (1, BLKN))
    aggrow = ag_ref[0:1, :] + ag_ref[1:2, :]
    orow = dinvrow * (aggrow + vrow) + b2_ref[...]
    o_ref[...] = jnp.transpose(orow, (1, 0))


_fin = pl.pallas_call(
    _fin_body,
    grid=(TGN,),
    in_specs=[
        pl.BlockSpec((2, BLKN), lambda i: (0, i)),
        pl.BlockSpec((BLKN,), lambda i: (i,)),
        pl.BlockSpec((BLKN,), lambda i: (i,)),
        pl.BlockSpec((1, 1), lambda i: (0, 0)),
    ],
    out_specs=pl.BlockSpec((BLKN, 1), lambda i: (i, 0)),
    out_shape=jax.ShapeDtypeStruct((NP, 1), jnp.float32),
)


def kernel(x, edge_index, W1, b1, W2, b2):
    e = edge_index.shape[1]
    pad_i = jnp.arange(EP - e, dtype=jnp.int32)
    dst = edge_index[1].astype(jnp.int32)
    pad_dst = TRASH + lax.rem(pad_i, jnp.int32(NP - N))
    dst2d = jnp.concatenate([dst, pad_dst]).reshape(NCH, CH)
    zeros_np = jnp.zeros((NP,), jnp.float32)
    degp = _deg_kernel(dst2d, zeros_np)

    ei2 = lax.optimization_barrier(edge_index)
    src = ei2[0].astype(jnp.int32)
    pad_src = lax.rem(pad_i, jnp.int32(N))
    src2d = jnp.concatenate([src, pad_src]).reshape(NCH, CH)

    x_p = jnp.concatenate(
        [x.astype(jnp.float32), jnp.zeros((NP - N, D), jnp.float32)]
    )
    u, dinv = _mm1(x_p, W1, degp)
    accp = _agg_kernel(src2d, dst2d, u)
    v = _mm2(accp, accp, u, dinv, W2, b1.reshape(1, H))
    aggp = _agg2_kernel(src2d, dst2d, v, zeros_np)
    out = _fin(aggp, v, dinv, b2.reshape(1, 1))
    return out[:N]


# speedup vs baseline: 3.1671x; 1.1220x over previous
import functools

import jax
import jax.numpy as jnp
from jax import lax
from jax.experimental import pallas as pl
from jax.experimental.pallas import tpu as pltpu
from jax.experimental.pallas import tpu_sc as plsc

N = 10000
D = 128
H = 128
NP = 10240
TRASH = 10000
NC = 2
NS = 16
L = 16
NW = NC * NS
CH = 128
CPT = 80
EP = NW * CPT * CH
NCH = EP // CH
BLKN = 1024
TGN = NP // BLKN
NPS = NP // NS

_mesh = plsc.VectorSubcoreMesh(
    core_axis_name="c", subcore_axis_name="s", num_cores=NC, num_subcores=NS
)


@functools.partial(
    pl.kernel,
    out_type=jax.ShapeDtypeStruct((NC, NP), jnp.float32),
    mesh=_mesh,
    scratch_types=[
        pltpu.VMEM((CPT, CH), jnp.int32),
        pltpu.VMEM((CH,), jnp.float32),
        pltpu.VMEM_SHARED((NP,), jnp.float32),
    ],
)
def _deg_kernel(dst2d, zeros_np, deg_out, didx, ones, deg_sp):
    cid = lax.axis_index("c")
    sid = lax.axis_index("s")
    wid = cid * NS + sid
    for j in range(CH // L):
        ones[pl.ds(j * L, L)] = jnp.ones((L,), jnp.float32)
    pltpu.sync_copy(dst2d.at[pl.ds(wid * CPT, CPT)], didx)

    @pl.when(sid == 0)
    def _():
        pltpu.sync_copy(zeros_np, deg_sp)

    plsc.subcore_barrier()

    def body(k, carry):
        pltpu.sync_copy(ones, deg_sp.at[didx.at[k]], add=True)
        return carry

    lax.fori_loop(0, CPT, body, 0)
    plsc.subcore_barrier()
    pltpu.sync_copy(
        deg_sp.at[pl.ds(sid * NPS, NPS)],
        deg_out.at[cid, pl.ds(sid * NPS, NPS)],
    )


GRP = 16
NGA = NCH // NW // GRP


@functools.partial(
    pl.kernel,
    out_type=jax.ShapeDtypeStruct((NC, NP, H), jnp.float32),
    mesh=_mesh,
    scratch_types=[
        pltpu.VMEM((2, GRP, CH), jnp.int32),
        pltpu.VMEM((2, GRP, CH), jnp.int32),
        pltpu.VMEM((2, CH, H), jnp.float32),
        pltpu.VMEM_SHARED((NP, H), jnp.float32),
        pltpu.SemaphoreType.DMA,
        pltpu.SemaphoreType.DMA,
        pltpu.SemaphoreType.DMA,
    ],
)
def _agg_kernel(src2d, dst2d, u_hbm, acc_out,
                sidx, didx, rows, acc_sp, sem0, sem1, isem):
    cid = lax.axis_index("c")
    sid = lax.axis_index("s")
    wid = cid * NS + sid
    start = wid * (NGA * GRP)

    pltpu.sync_copy(src2d.at[pl.ds(start, GRP)], sidx.at[0])
    pltpu.sync_copy(dst2d.at[pl.ds(start, GRP)], didx.at[0])

    @pl.when(sid == 0)
    def _():
        pltpu.sync_copy(u_hbm, acc_sp)

    plsc.subcore_barrier()

    def gbody(g, carry):
        p = lax.rem(g, 2)
        pnx = lax.rem(g + 1, 2)
        nbase = start + (g + 1) * GRP

        @pl.when(g + 1 < NGA)
        def _():
            pltpu.async_copy(src2d.at[pl.ds(nbase, GRP)], sidx.at[pnx], isem)
            pltpu.async_copy(dst2d.at[pl.ds(nbase, GRP)], didx.at[pnx], isem)

        sx = sidx.at[p]
        dx = didx.at[p]
        pltpu.async_copy(u_hbm.at[sx.at[0]], rows.at[0], sem0)

        def body(i, c2):
            k0 = i * 2
            pltpu.async_copy(u_hbm.at[sx.at[k0 + 1]], rows.at[1], sem1)
            pltpu.make_async_copy(
                u_hbm.at[sx.at[k0]], rows.at[0], sem0).wait()
            pltpu.sync_copy(rows.at[0], acc_sp.at[dx.at[k0]], add=True)

            @pl.when(k0 + 2 < GRP)
            def _():
                pltpu.async_copy(u_hbm.at[sx.at[k0 + 2]], rows.at[0], sem0)

            pltpu.make_async_copy(
                u_hbm.at[sx.at[k0 + 1]], rows.at[1], sem1).wait()
            pltpu.sync_copy(rows.at[1], acc_sp.at[dx.at[k0 + 1]], add=True)
            return c2

        lax.fori_loop(0, GRP // 2, body, 0)

        @pl.when(g + 1 < NGA)
        def _():
            pltpu.make_async_copy(
                src2d.at[pl.ds(nbase, GRP)], sidx.at[pnx], isem).wait()
            pltpu.make_async_copy(
                dst2d.at[pl.ds(nbase, GRP)], didx.at[pnx], isem).wait()

        return carry

    lax.fori_loop(0, NGA, gbody, 0)
    plsc.subcore_barrier()
    pltpu.sync_copy(
        acc_sp.at[pl.ds(sid * NPS, NPS)],
        acc_out.at[cid, pl.ds(sid * NPS, NPS)],
    )


@functools.partial(
    pl.kernel,
    out_type=jax.ShapeDtypeStruct((NC, NP), jnp.float32),
    mesh=_mesh,
    scratch_types=[
        pltpu.VMEM((CPT, CH), jnp.int32),
        pltpu.VMEM((CPT, CH), jnp.int32),
        pltpu.VMEM((8, CH), jnp.float32),
        pltpu.VMEM_SHARED((NP,), jnp.float32),
        pltpu.SemaphoreType.DMA((8,)),
        pltpu.SemaphoreType.DMA((8,)),
    ],
)
def _agg2_kernel(src2d, dst2d, v_hbm, zeros_np, agg_out,
                 sidx, didx, vals, agg_sp, gsem, ssem):
    cid = lax.axis_index("c")
    sid = lax.axis_index("s")
    wid = cid * NS + sid
    pltpu.sync_copy(src2d.at[pl.ds(wid * CPT, CPT)], sidx)
    pltpu.sync_copy(dst2d.at[pl.ds(wid * CPT, CPT)], didx)

    @pl.when(sid == 0)
    def _():
        pltpu.sync_copy(zeros_np, agg_sp)

    plsc.subcore_barrier()

    def wave(w, carry):
        k0 = w * 8
        for b in range(8):
            pltpu.async_copy(
                v_hbm.at[sidx.at[k0 + b]], vals.at[b], gsem.at[b])
        for b in range(8):
            pltpu.make_async_copy(
                v_hbm.at[sidx.at[k0 + b]], vals.at[b], gsem.at[b]).wait()
            pltpu.async_copy(
                vals.at[b], agg_sp.at[didx.at[k0 + b]], ssem.at[b],
                add=True)
        for b in range(8):
            pltpu.make_async_copy(
                vals.at[b], agg_sp.at[didx.at[k0 + b]], ssem.at[b]).wait()
        return carry

    lax.fori_loop(0, CPT // 8, wave, 0)
    plsc.subcore_barrier()
    pltpu.sync_copy(
        agg_sp.at[pl.ds(sid * NPS, NPS)],
        agg_out.at[cid, pl.ds(sid * NPS, NPS)],
    )


def _mm1_body(x_ref, w_ref, dg_ref, u_ref, dinv_ref):
    h = jnp.dot(x_ref[...], w_ref[...], preferred_element_type=jnp.float32)
    degrow = dg_ref[0:1, :] + dg_ref[1:2, :] + 1.0
    dinvrow = lax.rsqrt(degrow)
    dinvcol = jnp.transpose(dinvrow, (1, 0))
    u_ref[...] = h * dinvcol
    dinv_ref[...] = jnp.reshape(dinvrow, (BLKN,))


_mm1 = pl.pallas_call(
    _mm1_body,
    grid=(TGN,),
    in_specs=[
        pl.BlockSpec((BLKN, D), lambda i: (i, 0)),
        pl.BlockSpec((D, H), lambda i: (0, 0)),
        pl.BlockSpec((2, BLKN), lambda i: (0, i)),
    ],
    out_specs=[
        pl.BlockSpec((BLKN, H), lambda i: (i, 0)),
        pl.BlockSpec((BLKN,), lambda i: (i,)),
    ],
    out_shape=[
        jax.ShapeDtypeStruct((NP, H), jnp.float32),
        jax.ShapeDtypeStruct((NP,), jnp.float32),
    ],
)


def _mm2_body(a0_ref, a1_ref, u_ref, dinv_ref, w2_ref, b1_ref, v_ref):
    dinvrow = jnp.reshape(dinv_ref[...], (1, BLKN))
    dinvcol = jnp.transpose(dinvrow, (1, 0))
    acc = a0_ref[0] + a1_ref[0] - u_ref[...]
    a = jnp.maximum(dinvcol * acc + b1_ref[...], 0.0)
    zrow = lax.dot_general(
        w2_ref[...], a, (((0,), (1,)), ((), ())),
        preferred_element_type=jnp.float32)
    v_ref[...] = jnp.reshape(zrow * dinvrow, (BLKN,))


_mm2 = pl.pallas_call(
    _mm2_body,
    grid=(TGN,),
    in_specs=[
        pl.BlockSpec((1, BLKN, H), lambda i: (0, i, 0)),
        pl.BlockSpec((1, BLKN, H), lambda i: (1, i, 0)),
        pl.BlockSpec((BLKN, H), lambda i: (i, 0)),
        pl.BlockSpec((BLKN,), lambda i: (i,)),
        pl.BlockSpec((H, 1), lambda i: (0, 0)),
        pl.BlockSpec((1, H), lambda i: (0, 0)),
    ],
    out_specs=pl.BlockSpec((BLKN,), lambda i: (i,)),
    out_shape=jax.ShapeDtypeStruct((NP,), jnp.float32),
)


def _fin_body(ag_ref, v_ref, dinv_ref, b2_ref, o_ref):
    dinvrow = jnp.reshape(dinv_ref[...], (1, NP))[:, :N]
    vrow = jnp.reshape(v_ref[...], (1, NP))[:, :N]
    aggrow = ag_ref[0:1, :N] + ag_ref[1:2, :N]
    orow = dinvrow * (aggrow + vrow) + b2_ref[...]
    o_ref[...] = jnp.transpose(orow, (1, 0))


_fin = pl.pallas_call(
    _fin_body,
    grid=(1,),
    in_specs=[
        pl.BlockSpec((2, NP), lambda i: (0, 0)),
        pl.BlockSpec((NP,), lambda i: (0,)),
        pl.BlockSpec((NP,), lambda i: (0,)),
        pl.BlockSpec((1, 1), lambda i: (0, 0)),
    ],
    out_specs=pl.BlockSpec((N, 1), lambda i: (0, 0)),
    out_shape=jax.ShapeDtypeStruct((N, 1), jnp.float32),
)


def kernel(x, edge_index, W1, b1, W2, b2):
    e = edge_index.shape[1]
    pad_i = jnp.arange(EP - e, dtype=jnp.int32)
    dst = edge_index[1].astype(jnp.int32)
    pad_dst = TRASH + lax.rem(pad_i, jnp.int32(NP - N))
    dst2d = jnp.concatenate([dst, pad_dst]).reshape(NCH, CH)
    zeros_np = jnp.zeros((NP,), jnp.float32)
    degp = _deg_kernel(dst2d, zeros_np)

    ei2, _ = lax.optimization_barrier((edge_index, dst2d))
    src = ei2[0].astype(jnp.int32)
    pad_src = lax.rem(pad_i, jnp.int32(N))
    src2d = jnp.concatenate([src, pad_src]).reshape(NCH, CH)

    x_p = jnp.concatenate(
        [x.astype(jnp.float32), jnp.zeros((NP - N, D), jnp.float32)]
    )
    u, dinv = _mm1(x_p, W1, degp)
    accp = _agg_kernel(src2d, dst2d, u)
    v = _mm2(accp, accp, u, dinv, W2, b1.reshape(1, H))
    aggp = _agg2_kernel(src2d, dst2d, v, zeros_np)
    return _fin(aggp, v, dinv, b2.reshape(1, 1))
